# baseline (speedup 1.0000x reference)
import functools

import jax
import jax.numpy as jnp
from jax import lax
from jax.experimental import pallas as pl
from jax.experimental.pallas import tpu as pltpu
from jax.experimental.pallas import tpu_sc as plsc

B = 16384
S_DIM = 64
T_DIM = 64
R_DIM = S_DIM + T_DIM

NC = 2
NS = 16
NW = NC * NS
QPW = B // NW
CH = 128
NCH = QPW // CH

N_ENT_TBL = 12


def _sc_gather_body(hidx_hbm, tidx_hbm, ridx_hbm, *rest):
    tables = rest[:N_ENT_TBL]
    rels = rest[N_ENT_TBL:N_ENT_TBL + 3]
    outs = rest[N_ENT_TBL + 3:N_ENT_TBL + 3 + N_ENT_TBL]
    outs_r = rest[N_ENT_TBL + 3 + N_ENT_TBL:N_ENT_TBL + 3 + N_ENT_TBL + 3]
    scratch = rest[N_ENT_TBL + 3 + N_ENT_TBL + 3:]
    idxv, ridxv = scratch[0:2]
    bufs = scratch[2:2 + N_ENT_TBL]
    relbuf = scratch[2 + N_ENT_TBL]
    gsem, csem = scratch[3 + N_ENT_TBL:]

    cid = lax.axis_index("c")
    sid = lax.axis_index("s")
    wid = sid * NC + cid
    rowbase = wid * NCH

    pltpu.sync_copy(hidx_hbm.at[pl.ds(rowbase, NCH)], idxv.at[pl.ds(0, NCH)])
    pltpu.sync_copy(tidx_hbm.at[pl.ds(rowbase, NCH)], idxv.at[pl.ds(NCH, NCH)])
    pltpu.sync_copy(ridx_hbm.at[pl.ds(rowbase, NCH)], ridxv)

    @pl.loop(0, 2 * NCH)
    def _ent_chunk(j):
        off = (rowbase + j) * CH + jnp.where(j >= NCH, B - NCH * CH, 0)
        hs = [
            pltpu.async_copy(tables[k].at[idxv.at[j]], bufs[k], gsem)
            for k in range(N_ENT_TBL)
        ]
        for h in hs:
            h.wait()
        cs = [
            pltpu.async_copy(bufs[k], outs[k].at[pl.ds(off, CH)], csem)
            for k in range(N_ENT_TBL)
        ]
        for h in cs:
            h.wait()

    @pl.loop(0, NCH)
    def _rel_chunk(c):
        off = (rowbase + c) * CH
        for k in range(3):
            pltpu.async_copy(rels[k].at[ridxv.at[c]], relbuf, gsem).wait()
            pltpu.async_copy(relbuf, outs_r[k].at[pl.ds(off, CH)], csem).wait()


_SC_OUT = (
    [jax.ShapeDtypeStruct((2 * B, S_DIM), jnp.float32)] * N_ENT_TBL
    + [jax.ShapeDtypeStruct((B, R_DIM), jnp.float32)] * 3
)


@functools.cache
def _get_sc_gather():
    return pl.kernel(
        _sc_gather_body,
        out_type=tuple(_SC_OUT),
        mesh=plsc.VectorSubcoreMesh(
            core_axis_name="c", subcore_axis_name="s",
            num_cores=NC, num_subcores=NS,
        ),
        scratch_types=(
            [pltpu.VMEM((2 * NCH, CH), jnp.int32),
             pltpu.VMEM((NCH, CH), jnp.int32)]
            + [pltpu.VMEM((CH, S_DIM), jnp.float32)] * N_ENT_TBL
            + [pltpu.VMEM((CH, R_DIM), jnp.float32),
               pltpu.SemaphoreType.DMA, pltpu.SemaphoreType.DMA]
        ),
        compiler_params=pltpu.CompilerParams(use_tc_tiling_on_sc=False),
    )


def _tc_math_body(yy_ref, mm_ref, dd_ref,
                  hx_ref, hy_ref, hz_ref, hyf, hyp, hya, hmf, hmp, hma, hdf, hdp, hda,
                  tx_ref, ty_ref, tz_ref, tyf, typ, tya, tmf, tmp_, tma, tdf, tdp, tda,
                  rw_ref, rx_ref, rz_ref, o_ref):
    yy = yy_ref[:]
    mm = mm_ref[:]
    dd = dd_ref[:]

    h_time = (hya[:] * jnp.sin(hyf[:] * yy + hyp[:])
              + hma[:] * jnp.sin(hmf[:] * mm + hmp[:])
              + hda[:] * jnp.sin(hdf[:] * dd + hdp[:]))
    t_time = (tya[:] * jnp.sin(tyf[:] * yy + typ[:])
              + tma[:] * jnp.sin(tmf[:] * mm + tmp_[:])
              + tda[:] * jnp.sin(tdf[:] * dd + tdp[:]))

    h_x = jnp.concatenate([hx_ref[:], h_time], axis=1)
    h_y = jnp.concatenate([hy_ref[:], h_time], axis=1)
    h_z = jnp.concatenate([hz_ref[:], h_time], axis=1)
    t_x = jnp.concatenate([tx_ref[:], t_time], axis=1)
    t_y = jnp.concatenate([ty_ref[:], t_time], axis=1)
    t_z = jnp.concatenate([tz_ref[:], t_time], axis=1)

    r_w = rw_ref[:]
    r_x = rx_ref[:]
    r_z = rz_ref[:]
    rel_y = t_y

    denom = jnp.sqrt(r_w ** 2 + r_x ** 2 + rel_y ** 2 + r_z ** 2)
    w = r_w / denom
    x = r_x / denom
    y = rel_y / denom
    z = r_z / denom

    ct_x = (1 - 2 * y * y - 2 * z * z) * h_x + (2 * x * y - 2 * z * w) * h_y + (2 * x * z + 2 * y * w) * h_z
    ct_y = (2 * x * y + 2 * z * w) * h_x + (1 - 2 * x * x - 2 * z * z) * h_y + (2 * y * z - 2 * x * w) * h_z
    ct_z = (2 * x * z - 2 * y * w) * h_x + (2 * y * z + 2 * x * w) * h_y + (1 - 2 * x * x - 2 * y * y) * h_z
    score1 = jnp.sqrt((ct_x - t_x) ** 2 + (ct_y - t_y) ** 2 + (ct_z - t_z) ** 2)

    x = -x
    y = -y
    z = -z
    ch_x = (1 - 2 * y * y - 2 * z * z) * t_x + (2 * x * y - 2 * z * w) * t_y + (2 * x * z + 2 * y * w) * t_z
    ch_y = (2 * x * y + 2 * z * w) * t_x + (1 - 2 * x * x - 2 * z * z) * t_y + (2 * y * z - 2 * x * w) * t_z
    ch_z = (2 * x * z - 2 * y * w) * t_x + (2 * y * z + 2 * x * w) * t_y + (1 - 2 * x * x - 2 * y * y) * t_z
    score2 = jnp.sqrt((ch_x - h_x) ** 2 + (ch_y - h_y) ** 2 + (ch_z - h_z) ** 2)

    s1 = score1.mean(axis=1)
    s2 = score2.mean(axis=1)
    o_ref[:] = (12.0 - (s1 + s2) / 2.0)[:, None]


TC_R = 1024


def _tc_math(yy, mm, dd, ent_gathered, rel_gathered):
    grid = B // TC_R

    def bs(d, tail=False):
        if tail:
            return pl.BlockSpec((TC_R, d), lambda i: (i + B // TC_R, 0))
        return pl.BlockSpec((TC_R, d), lambda i: (i, 0))

    in_specs = (
        [bs(1)] * 3
        + [bs(S_DIM)] * N_ENT_TBL
        + [bs(S_DIM, tail=True)] * N_ENT_TBL
        + [bs(R_DIM)] * 3
    )
    return pl.pallas_call(
        _tc_math_body,
        grid=(grid,),
        in_specs=in_specs,
        out_specs=bs(1),
        out_shape=jax.ShapeDtypeStruct((B, 1), jnp.float32),
    )(yy, mm, dd, *ent_gathered, *ent_gathered, *rel_gathered)


def kernel(heads, rels, tails, years, months, days, ent_x, ent_y, ent_z,
           rel_w_t, rel_x_t, rel_y_t, rel_z_t,
           y_freq, y_phi, y_amp, m_freq, m_phi, m_amp, d_freq, d_phi, d_amp):
    hidx = heads.astype(jnp.int32).reshape(NW * NCH, CH)
    tidx = tails.astype(jnp.int32).reshape(NW * NCH, CH)
    ridx = rels.astype(jnp.int32).reshape(NW * NCH, CH)

    gathered = _get_sc_gather()(
        hidx, tidx, ridx,
        ent_x, ent_y, ent_z, y_freq, y_phi, y_amp,
        m_freq, m_phi, m_amp, d_freq, d_phi, d_amp,
        rel_w_t, rel_x_t, rel_z_t,
    )

    out2d = _tc_math(
        years.reshape(B, 1), months.reshape(B, 1), days.reshape(B, 1),
        gathered[:N_ENT_TBL], gathered[N_ENT_TBL:],
    )
    return out2d.reshape(B)


# speedup vs baseline: 1.3230x; 1.3230x over previous
import functools

import jax
import jax.numpy as jnp
from jax import lax
from jax.experimental import pallas as pl
from jax.experimental.pallas import tpu as pltpu
from jax.experimental.pallas import tpu_sc as plsc

B = 16384
S_DIM = 64
T_DIM = 64
R_DIM = S_DIM + T_DIM

NC = 2
NS = 16
NW = NC * NS
QPW = B // NW
CH = 128
NCH = QPW // CH

N_PAIR = 6


def _sc_gather_body(hidx_hbm, ridx_hbm, *rest):
    pairs = rest[:N_PAIR]
    rels = rest[N_PAIR:N_PAIR + 3]
    outs = rest[N_PAIR + 3:N_PAIR + 3 + N_PAIR]
    outs_r = rest[N_PAIR + 3 + N_PAIR:N_PAIR + 3 + N_PAIR + 3]
    scratch = rest[N_PAIR + 3 + N_PAIR + 3:]
    idxv, ridxv = scratch[0:2]
    bufs = scratch[2:2 + N_PAIR]
    gsem, csem = scratch[2 + N_PAIR:]

    cid = lax.axis_index("c")
    sid = lax.axis_index("s")
    wid = sid * NC + cid
    rowbase = wid * NCH

    pltpu.sync_copy(hidx_hbm.at[wid], idxv)
    pltpu.sync_copy(ridx_hbm.at[wid], ridxv)

    @pl.loop(0, 2 * NCH)
    def _ent_chunk(j):
        off = (rowbase + j) * CH + jnp.where(j >= NCH, B - NCH * CH, 0)
        hs = [
            pltpu.async_copy(pairs[k].at[idxv.at[j]], bufs[k], gsem)
            for k in range(N_PAIR)
        ]
        for h in hs:
            h.wait()
        cs = [
            pltpu.async_copy(bufs[k], outs[k].at[pl.ds(off, CH)], csem)
            for k in range(N_PAIR)
        ]
        for h in cs:
            h.wait()

    @pl.loop(0, NCH)
    def _rel_chunk(c):
        off = (rowbase + c) * CH
        hs = [
            pltpu.async_copy(rels[k].at[ridxv.at[c]], bufs[k], gsem)
            for k in range(3)
        ]
        for h in hs:
            h.wait()
        cs = [
            pltpu.async_copy(bufs[k], outs_r[k].at[pl.ds(off, CH)], csem)
            for k in range(3)
        ]
        for h in cs:
            h.wait()


_SC_OUT = (
    [jax.ShapeDtypeStruct((2 * B, R_DIM), jnp.float32)] * N_PAIR
    + [jax.ShapeDtypeStruct((B, R_DIM), jnp.float32)] * 3
)


@functools.cache
def _get_sc_gather():
    return pl.kernel(
        _sc_gather_body,
        out_type=tuple(_SC_OUT),
        mesh=plsc.VectorSubcoreMesh(
            core_axis_name="c", subcore_axis_name="s",
            num_cores=NC, num_subcores=NS,
        ),
        scratch_types=(
            [pltpu.VMEM((2 * NCH, CH), jnp.int32),
             pltpu.VMEM((NCH, CH), jnp.int32)]
            + [pltpu.VMEM((CH, R_DIM), jnp.float32)] * N_PAIR
            + [pltpu.SemaphoreType.DMA, pltpu.SemaphoreType.DMA]
        ),
        compiler_params=pltpu.CompilerParams(use_tc_tiling_on_sc=True),
    )


def _tc_math_body(yy_ref, mm_ref, dd_ref,
                  h0, h1, h2, h3, h4, h5,
                  t0, t1, t2, t3, t4, t5,
                  rw_ref, rx_ref, rz_ref, o_ref):
    yy = yy_ref[:]
    mm = mm_ref[:]
    dd = dd_ref[:]

    def split(p):
        return p[:, :S_DIM], p[:, S_DIM:]

    hx, hy = split(h0[:])
    hz, hyf = split(h1[:])
    hyp, hya = split(h2[:])
    hmf, hmp = split(h3[:])
    hma, hdf = split(h4[:])
    hdp, hda = split(h5[:])
    tx, ty = split(t0[:])
    tz, tyf = split(t1[:])
    typ, tya = split(t2[:])
    tmf, tmp_ = split(t3[:])
    tma, tdf = split(t4[:])
    tdp, tda = split(t5[:])

    h_time = (hya * jnp.sin(hyf * yy + hyp)
              + hma * jnp.sin(hmf * mm + hmp)
              + hda * jnp.sin(hdf * dd + hdp))
    t_time = (tya * jnp.sin(tyf * yy + typ)
              + tma * jnp.sin(tmf * mm + tmp_)
              + tda * jnp.sin(tdf * dd + tdp))

    h_x = jnp.concatenate([hx, h_time], axis=1)
    h_y = jnp.concatenate([hy, h_time], axis=1)
    h_z = jnp.concatenate([hz, h_time], axis=1)
    t_x = jnp.concatenate([tx, t_time], axis=1)
    t_y = jnp.concatenate([ty, t_time], axis=1)
    t_z = jnp.concatenate([tz, t_time], axis=1)

    r_w = rw_ref[:]
    r_x = rx_ref[:]
    r_z = rz_ref[:]
    rel_y = t_y

    denom = jnp.sqrt(r_w ** 2 + r_x ** 2 + rel_y ** 2 + r_z ** 2)
    w = r_w / denom
    x = r_x / denom
    y = rel_y / denom
    z = r_z / denom

    ct_x = (1 - 2 * y * y - 2 * z * z) * h_x + (2 * x * y - 2 * z * w) * h_y + (2 * x * z + 2 * y * w) * h_z
    ct_y = (2 * x * y + 2 * z * w) * h_x + (1 - 2 * x * x - 2 * z * z) * h_y + (2 * y * z - 2 * x * w) * h_z
    ct_z = (2 * x * z - 2 * y * w) * h_x + (2 * y * z + 2 * x * w) * h_y + (1 - 2 * x * x - 2 * y * y) * h_z
    score1 = jnp.sqrt((ct_x - t_x) ** 2 + (ct_y - t_y) ** 2 + (ct_z - t_z) ** 2)

    x = -x
    y = -y
    z = -z
    ch_x = (1 - 2 * y * y - 2 * z * z) * t_x + (2 * x * y - 2 * z * w) * t_y + (2 * x * z + 2 * y * w) * t_z
    ch_y = (2 * x * y + 2 * z * w) * t_x + (1 - 2 * x * x - 2 * z * z) * t_y + (2 * y * z - 2 * x * w) * t_z
    ch_z = (2 * x * z - 2 * y * w) * t_x + (2 * y * z + 2 * x * w) * t_y + (1 - 2 * x * x - 2 * y * y) * t_z
    score2 = jnp.sqrt((ch_x - h_x) ** 2 + (ch_y - h_y) ** 2 + (ch_z - h_z) ** 2)

    s1 = score1.mean(axis=1)
    s2 = score2.mean(axis=1)
    o_ref[:] = (12.0 - (s1 + s2) / 2.0)[:, None]


TC_R = 1024


def _tc_math(yy, mm, dd, ent_gathered, rel_gathered):
    grid = B // TC_R

    def bs(d, tail=False):
        if tail:
            return pl.BlockSpec((TC_R, d), lambda i: (i + B // TC_R, 0))
        return pl.BlockSpec((TC_R, d), lambda i: (i, 0))

    in_specs = (
        [bs(1)] * 3
        + [bs(R_DIM)] * N_PAIR
        + [bs(R_DIM, tail=True)] * N_PAIR
        + [bs(R_DIM)] * 3
    )
    return pl.pallas_call(
        _tc_math_body,
        grid=(grid,),
        in_specs=in_specs,
        out_specs=bs(1),
        out_shape=jax.ShapeDtypeStruct((B, 1), jnp.float32),
    )(yy, mm, dd, *ent_gathered, *ent_gathered, *rel_gathered)


def kernel(heads, rels, tails, years, months, days, ent_x, ent_y, ent_z,
           rel_w_t, rel_x_t, rel_y_t, rel_z_t,
           y_freq, y_phi, y_amp, m_freq, m_phi, m_amp, d_freq, d_phi, d_amp):
    hh = heads.astype(jnp.int32).reshape(NW, NCH, CH)
    tt = tails.astype(jnp.int32).reshape(NW, NCH, CH)
    hidx = jnp.concatenate([hh, tt], axis=1)
    ridx = rels.astype(jnp.int32).reshape(NW, NCH, CH)

    pairs = [
        jnp.concatenate(p, axis=1)
        for p in ((ent_x, ent_y), (ent_z, y_freq), (y_phi, y_amp),
                  (m_freq, m_phi), (m_amp, d_freq), (d_phi, d_amp))
    ]

    gathered = _get_sc_gather()(
        hidx, ridx, *pairs, rel_w_t, rel_x_t, rel_z_t,
    )

    out2d = _tc_math(
        years.reshape(B, 1), months.reshape(B, 1), days.reshape(B, 1),
        gathered[:N_PAIR], gathered[N_PAIR:],
    )
    return out2d.reshape(B)


# speedup vs baseline: 1.8224x; 1.3775x over previous
import functools

import jax
import jax.numpy as jnp
from jax import lax
from jax.experimental import pallas as pl
from jax.experimental.pallas import tpu as pltpu
from jax.experimental.pallas import tpu_sc as plsc

B = 16384
S_DIM = 64
T_DIM = 64
R_DIM = S_DIM + T_DIM

NC = 2
NS = 16
NW = NC * NS
QPW = B // NW
CH = 128
NCH = QPW // CH

N_PAIR = 6


def _sc_gather_body(hidx_hbm, ridx_hbm, *rest):
    pairs = rest[:N_PAIR]
    rels = rest[N_PAIR:N_PAIR + 3]
    outs = rest[N_PAIR + 3:N_PAIR + 3 + N_PAIR]
    outs_r = rest[N_PAIR + 3 + N_PAIR:N_PAIR + 3 + N_PAIR + 3]
    scratch = rest[N_PAIR + 3 + N_PAIR + 3:]
    idxv, ridxv = scratch[0:2]
    bufs = scratch[2:2 + N_PAIR]
    gsem, csem = scratch[2 + N_PAIR:]

    cid = lax.axis_index("c")
    sid = lax.axis_index("s")
    wid = sid * NC + cid
    rowbase = wid * NCH

    pltpu.sync_copy(hidx_hbm.at[wid], idxv)
    pltpu.sync_copy(ridx_hbm.at[wid], ridxv)

    @pl.loop(0, 2 * NCH)
    def _ent_chunk(j):
        off = (rowbase + j) * CH + jnp.where(j >= NCH, B - NCH * CH, 0)
        hs = [
            pltpu.async_copy(pairs[k].at[idxv.at[j]], bufs[k], gsem)
            for k in range(N_PAIR)
        ]
        for h in hs:
            h.wait()
        cs = [
            pltpu.async_copy(bufs[k], outs[k].at[pl.ds(off, CH)], csem)
            for k in range(N_PAIR)
        ]
        for h in cs:
            h.wait()

    @pl.loop(0, NCH)
    def _rel_chunk(c):
        off = (rowbase + c) * CH
        hs = [
            pltpu.async_copy(rels[k].at[ridxv.at[c]], bufs[k], gsem)
            for k in range(3)
        ]
        for h in hs:
            h.wait()
        cs = [
            pltpu.async_copy(bufs[k], outs_r[k].at[pl.ds(off, CH)], csem)
            for k in range(3)
        ]
        for h in cs:
            h.wait()


_SC_OUT = (
    [jax.ShapeDtypeStruct((2 * B, R_DIM), jnp.float32)] * N_PAIR
    + [jax.ShapeDtypeStruct((B, R_DIM), jnp.float32)] * 3
)


@functools.cache
def _get_sc_gather():
    return pl.kernel(
        _sc_gather_body,
        out_type=tuple(_SC_OUT),
        mesh=plsc.VectorSubcoreMesh(
            core_axis_name="c", subcore_axis_name="s",
            num_cores=NC, num_subcores=NS,
        ),
        scratch_types=(
            [pltpu.VMEM((2 * NCH, CH), jnp.int32),
             pltpu.VMEM((NCH, CH), jnp.int32)]
            + [pltpu.VMEM((CH, R_DIM), jnp.float32)] * N_PAIR
            + [pltpu.SemaphoreType.DMA, pltpu.SemaphoreType.DMA]
        ),
        compiler_params=pltpu.CompilerParams(use_tc_tiling_on_sc=True),
    )


def _tc_math_body(yy_ref, mm_ref, dd_ref,
                  h0, h1, h2, h3, h4, h5,
                  t0, t1, t2, t3, t4, t5,
                  rw_ref, rx_ref, rz_ref, o_ref):
    yy = yy_ref[:]
    mm = mm_ref[:]
    dd = dd_ref[:]

    def split(p):
        return p[:, :S_DIM], p[:, S_DIM:]

    hx, hy = split(h0[:])
    hz, hyf = split(h1[:])
    hyp, hya = split(h2[:])
    hmf, hmp = split(h3[:])
    hma, hdf = split(h4[:])
    hdp, hda = split(h5[:])
    tx, ty = split(t0[:])
    tz, tyf = split(t1[:])
    typ, tya = split(t2[:])
    tmf, tmp_ = split(t3[:])
    tma, tdf = split(t4[:])
    tdp, tda = split(t5[:])

    h_time = (hya * jnp.sin(hyf * yy + hyp)
              + hma * jnp.sin(hmf * mm + hmp)
              + hda * jnp.sin(hdf * dd + hdp))
    t_time = (tya * jnp.sin(tyf * yy + typ)
              + tma * jnp.sin(tmf * mm + tmp_)
              + tda * jnp.sin(tdf * dd + tdp))

    h_x = jnp.concatenate([hx, h_time], axis=1)
    h_y = jnp.concatenate([hy, h_time], axis=1)
    h_z = jnp.concatenate([hz, h_time], axis=1)
    t_x = jnp.concatenate([tx, t_time], axis=1)
    t_y = jnp.concatenate([ty, t_time], axis=1)
    t_z = jnp.concatenate([tz, t_time], axis=1)

    r_w = rw_ref[:]
    r_x = rx_ref[:]
    r_z = rz_ref[:]
    rel_y = t_y

    denom = jnp.sqrt(r_w ** 2 + r_x ** 2 + rel_y ** 2 + r_z ** 2)
    w = r_w / denom
    x = r_x / denom
    y = rel_y / denom
    z = r_z / denom

    ct_x = (1 - 2 * y * y - 2 * z * z) * h_x + (2 * x * y - 2 * z * w) * h_y + (2 * x * z + 2 * y * w) * h_z
    ct_y = (2 * x * y + 2 * z * w) * h_x + (1 - 2 * x * x - 2 * z * z) * h_y + (2 * y * z - 2 * x * w) * h_z
    ct_z = (2 * x * z - 2 * y * w) * h_x + (2 * y * z + 2 * x * w) * h_y + (1 - 2 * x * x - 2 * y * y) * h_z
    score1 = jnp.sqrt((ct_x - t_x) ** 2 + (ct_y - t_y) ** 2 + (ct_z - t_z) ** 2)

    x = -x
    y = -y
    z = -z
    ch_x = (1 - 2 * y * y - 2 * z * z) * t_x + (2 * x * y - 2 * z * w) * t_y + (2 * x * z + 2 * y * w) * t_z
    ch_y = (2 * x * y + 2 * z * w) * t_x + (1 - 2 * x * x - 2 * z * z) * t_y + (2 * y * z - 2 * x * w) * t_z
    ch_z = (2 * x * z - 2 * y * w) * t_x + (2 * y * z + 2 * x * w) * t_y + (1 - 2 * x * x - 2 * y * y) * t_z
    score2 = jnp.sqrt((ch_x - h_x) ** 2 + (ch_y - h_y) ** 2 + (ch_z - h_z) ** 2)

    s1 = score1.mean(axis=1)
    s2 = score2.mean(axis=1)
    o_ref[:] = (12.0 - (s1 + s2) / 2.0)[:, None]


NUM_ENT = 100000
RP_E = 1024


def _tc_repack_body(*refs):
    ins = refs[:2 * N_PAIR]
    outs = refs[2 * N_PAIR:]
    for k in range(N_PAIR):
        a = jnp.transpose(ins[2 * k][:], (1, 0))
        b = jnp.transpose(ins[2 * k + 1][:], (1, 0))
        outs[k][:] = jnp.concatenate([a, b], axis=1)


def _tc_repack(vts):
    grid = (NUM_ENT + RP_E - 1) // RP_E
    return pl.pallas_call(
        _tc_repack_body,
        grid=(grid,),
        in_specs=[pl.BlockSpec((S_DIM, RP_E), lambda i: (0, i))] * (2 * N_PAIR),
        out_specs=[pl.BlockSpec((RP_E, R_DIM), lambda i: (i, 0))] * N_PAIR,
        out_shape=[jax.ShapeDtypeStruct((NUM_ENT, R_DIM), jnp.float32)] * N_PAIR,
    )(*vts)


TC_R = 1024


def _tc_math(yy, mm, dd, ent_gathered, rel_gathered):
    grid = B // TC_R

    def bs(d, tail=False):
        if tail:
            return pl.BlockSpec((TC_R, d), lambda i: (i + B // TC_R, 0))
        return pl.BlockSpec((TC_R, d), lambda i: (i, 0))

    in_specs = (
        [bs(1)] * 3
        + [bs(R_DIM)] * N_PAIR
        + [bs(R_DIM, tail=True)] * N_PAIR
        + [bs(R_DIM)] * 3
    )
    return pl.pallas_call(
        _tc_math_body,
        grid=(grid,),
        in_specs=in_specs,
        out_specs=bs(1),
        out_shape=jax.ShapeDtypeStruct((B, 1), jnp.float32),
    )(yy, mm, dd, *ent_gathered, *ent_gathered, *rel_gathered)


def kernel(heads, rels, tails, years, months, days, ent_x, ent_y, ent_z,
           rel_w_t, rel_x_t, rel_y_t, rel_z_t,
           y_freq, y_phi, y_amp, m_freq, m_phi, m_amp, d_freq, d_phi, d_amp):
    hh = heads.astype(jnp.int32).reshape(NW, NCH, CH)
    tt = tails.astype(jnp.int32).reshape(NW, NCH, CH)
    hidx = jnp.concatenate([hh, tt], axis=1)
    ridx = rels.astype(jnp.int32).reshape(NW, NCH, CH)

    pairs = _tc_repack([
        t.T for t in (ent_x, ent_y, ent_z, y_freq, y_phi, y_amp,
                      m_freq, m_phi, m_amp, d_freq, d_phi, d_amp)
    ])

    gathered = _get_sc_gather()(
        hidx, ridx, *pairs, rel_w_t, rel_x_t, rel_z_t,
    )

    out2d = _tc_math(
        years.reshape(B, 1), months.reshape(B, 1), days.reshape(B, 1),
        gathered[:N_PAIR], gathered[N_PAIR:],
    )
    return out2d.reshape(B)
